# baseline (speedup 1.0000x reference)
import functools

import jax
import jax.numpy as jnp
from jax import lax
from jax.experimental import pallas as pl
from jax.experimental.pallas import tpu as pltpu
from jax.experimental.pallas import tpu_sc as plsc

_B, _N, _F, _E = 1024, 500, 64, 16
_TN = 25
_STEPS = _N // _TN
_ROWS = _TN * _E

_SCW = 25
_PER_W = (2 * _N) // _SCW
_SLAB = 8
_NDMA = _PER_W // _SLAB


def _tc_body(xt_ref, wt_ref, b16_ref, wvp_ref, bv_ref,
             val_ref, z_ref, bt_ref):
    i = pl.program_id(0)

    @pl.when(i == 0)
    def _init():
        val_ref[...] = jnp.full((1, _B), bv_ref[0, 0], jnp.float32)
        e_idx = jax.lax.broadcasted_iota(jnp.int32, (_E, 1), 0)
        bt = jnp.zeros((_E, 1), jnp.float32)
        for e in range(_E):
            bt = jnp.where(e_idx == e, b16_ref[0, e], bt)
        bt_ref[...] = bt

    wt = wt_ref[...]
    bt = bt_ref[...]
    for n in range(_TN):
        y = jnp.dot(wt, xt_ref[n], preferred_element_type=jnp.float32)
        z_ref[n * _E:(n + 1) * _E, :] = jnp.maximum(y + bt, 0.0)
    part = jnp.dot(wvp_ref[0], z_ref[...],
                   preferred_element_type=jnp.float32)
    val_ref[...] += part


_sc_mesh = plsc.VectorSubcoreMesh(core_axis_name="c", subcore_axis_name="s")


@functools.partial(
    pl.kernel,
    mesh=_sc_mesh,
    out_type=jax.ShapeDtypeStruct((2 * _N, _B), jnp.float32),
    scratch_types=[
        pltpu.VMEM((32,), jnp.float32),
        pltpu.VMEM((_SLAB, _B), jnp.float32),
        pltpu.SemaphoreType.DMA,
    ],
)
def _sc_actions(p32_hbm, out_hbm, p_v, slab_v, sem):
    wid = lax.axis_index("c") * 16 + lax.axis_index("s")

    @pl.when(wid < _SCW)
    def _work():
        pltpu.sync_copy(p32_hbm, p_v)
        v0 = p_v[pl.ds(0, 16)]
        v1 = p_v[pl.ds(16, 16)]
        a0v = 1.0 / (1.0 + jnp.exp(-v0))
        a1v = 0.5 / (1.0 + jnp.exp(-v1))
        for r in range(_SLAB):
            rv = a0v if r % 2 == 0 else a1v
            for t in range(_B // 16):
                slab_v[r, pl.ds(16 * t, 16)] = rv
        base = wid * _PER_W
        cps = [
            pltpu.make_async_copy(
                slab_v, out_hbm.at[pl.ds(base + j * _SLAB, _SLAB)], sem)
            for j in range(_NDMA)
        ]
        for c in cps:
            c.start()
        for c in cps:
            c.wait()


def kernel(node_features_gen, W_embed, b_embed, param, W_val, b_val):
    actt = _sc_actions(jnp.repeat(param, 16))

    xt = jnp.transpose(node_features_gen, (1, 2, 0))
    wt = W_embed.T
    wvp = W_val.reshape(_STEPS, 1, _ROWS)

    val = pl.pallas_call(
        _tc_body,
        grid=(_STEPS,),
        in_specs=[
            pl.BlockSpec((_TN, _F, _B), lambda i: (i, 0, 0)),
            pl.BlockSpec((_E, _F), lambda i: (0, 0)),
            pl.BlockSpec((1, _E), lambda i: (0, 0)),
            pl.BlockSpec((1, 1, _ROWS), lambda i: (i, 0, 0)),
            pl.BlockSpec((1, 1), lambda i: (0, 0)),
        ],
        out_specs=pl.BlockSpec((1, _B), lambda i: (0, 0)),
        out_shape=jax.ShapeDtypeStruct((1, _B), jnp.float32),
        scratch_shapes=[pltpu.VMEM((_ROWS, _B), jnp.float32),
                        pltpu.VMEM((_E, 1), jnp.float32)],
    )(xt, wt, b_embed.reshape(1, _E), wvp, b_val.reshape(1, 1))
    return actt.T, val.reshape(_B)


# speedup vs baseline: 1.3629x; 1.3629x over previous
import jax
import jax.numpy as jnp
from jax.experimental import pallas as pl
from jax.experimental.pallas import tpu as pltpu

_B, _N, _F, _E = 1024, 500, 64, 16
_TN = 25
_STEPS = _N // _TN
_ROWS = _TN * _E
_LANES = 128


def _fused_body(xt_ref, wt_ref, b16_ref, wvp_ref, p_ref, bv_ref,
                actt_ref, val_ref, z_ref, bt_ref):
    i = pl.program_id(0)

    @pl.when(i == 0)
    def _init():
        val_ref[...] = jnp.full((1, _B), bv_ref[0, 0], jnp.float32)
        e_idx = jax.lax.broadcasted_iota(jnp.int32, (_E, 1), 0)
        bt = jnp.zeros((_E, 1), jnp.float32)
        for e in range(_E):
            bt = jnp.where(e_idx == e, b16_ref[0, e], bt)
        bt_ref[...] = bt

    a0 = jax.nn.sigmoid(p_ref[0, 0])
    a1 = jax.nn.sigmoid(p_ref[0, 1]) * 0.5
    r = jax.lax.broadcasted_iota(jnp.int32, (2 * _N, _LANES), 0)
    actt_ref[...] = jnp.where(r % 2 == 0, a0, a1)

    wt = wt_ref[...]
    bt = bt_ref[...]
    for n in range(_TN):
        y = jnp.dot(wt, xt_ref[n], preferred_element_type=jnp.float32)
        z_ref[n * _E:(n + 1) * _E, :] = jnp.maximum(y + bt, 0.0)
    part = jnp.dot(wvp_ref[0], z_ref[...],
                   preferred_element_type=jnp.float32)
    val_ref[...] += part


def kernel(node_features_gen, W_embed, b_embed, param, W_val, b_val):
    xt = jnp.transpose(node_features_gen, (1, 2, 0))
    wt = W_embed.T
    wvp = W_val.T.reshape(_STEPS, 1, _ROWS)

    actt, val = pl.pallas_call(
        _fused_body,
        grid=(_STEPS,),
        in_specs=[
            pl.BlockSpec((_TN, _F, _B), lambda i: (i, 0, 0)),
            pl.BlockSpec((_E, _F), lambda i: (0, 0)),
            pl.BlockSpec((1, _E), lambda i: (0, 0)),
            pl.BlockSpec((1, 1, _ROWS), lambda i: (i, 0, 0)),
            pl.BlockSpec((1, 2), lambda i: (0, 0)),
            pl.BlockSpec((1, 1), lambda i: (0, 0)),
        ],
        out_specs=[
            pl.BlockSpec((2 * _N, _LANES),
                         lambda i: (0, jnp.minimum(i, _B // _LANES - 1))),
            pl.BlockSpec((1, _B), lambda i: (0, 0)),
        ],
        out_shape=[
            jax.ShapeDtypeStruct((2 * _N, _B), jnp.float32),
            jax.ShapeDtypeStruct((1, _B), jnp.float32),
        ],
        scratch_shapes=[pltpu.VMEM((_ROWS, _B), jnp.float32),
                        pltpu.VMEM((_E, 1), jnp.float32)],
    )(xt, wt, b_embed.reshape(1, _E), wvp,
      param.reshape(1, 2), b_val.reshape(1, 1))
    return actt.T, val.reshape(_B)


# speedup vs baseline: 1.4040x; 1.0302x over previous
import jax
import jax.numpy as jnp
from jax.experimental import pallas as pl
from jax.experimental.pallas import tpu as pltpu

_B, _N, _F, _E = 1024, 500, 64, 16
_TN = 25
_STEPS = _N // _TN
_ROWS = _TN * _E
_LANES = 128


def _fused_body(xt_ref, wt_ref, b16_ref, wvf_ref, p_ref, bv_ref,
                actt_ref, val_ref, z_ref, bt_ref):
    i = pl.program_id(0)

    @pl.when(i == 0)
    def _init():
        val_ref[...] = jnp.full((1, _B), bv_ref[0, 0], jnp.float32)
        e_idx = jax.lax.broadcasted_iota(jnp.int32, (_E, 1), 0)
        bt = jnp.zeros((_E, 1), jnp.float32)
        for e in range(_E):
            bt = jnp.where(e_idx == e, b16_ref[0, e], bt)
        bt_ref[...] = bt

    a0 = jax.nn.sigmoid(p_ref[0, 0])
    a1 = jax.nn.sigmoid(p_ref[0, 1]) * 0.5
    r = jax.lax.broadcasted_iota(jnp.int32, (2 * _N, _LANES), 0)
    actt_ref[...] = jnp.where(r % 2 == 0, a0, a1)

    wt = wt_ref[...]
    bt = bt_ref[...]
    for n in range(_TN):
        y = jnp.dot(wt, xt_ref[n], preferred_element_type=jnp.float32)
        z_ref[n * _E:(n + 1) * _E, :] = jnp.maximum(y + bt, 0.0)
    z = z_ref[...]
    wv = wvf_ref[...]
    for k in range(_STEPS):
        @pl.when(i == k)
        def _step_dot(k=k):
            part = jnp.dot(wv[:, k * _ROWS:(k + 1) * _ROWS], z,
                           preferred_element_type=jnp.float32)
            val_ref[...] += part


def kernel(node_features_gen, W_embed, b_embed, param, W_val, b_val):
    xt = jnp.transpose(node_features_gen, (1, 2, 0))
    wt = W_embed.T

    actt, val = pl.pallas_call(
        _fused_body,
        grid=(_STEPS,),
        in_specs=[
            pl.BlockSpec((_TN, _F, _B), lambda i: (i, 0, 0)),
            pl.BlockSpec((_E, _F), lambda i: (0, 0)),
            pl.BlockSpec((1, _E), lambda i: (0, 0)),
            pl.BlockSpec((1, _N * _E), lambda i: (0, 0)),
            pl.BlockSpec((1, 2), lambda i: (0, 0)),
            pl.BlockSpec((1, 1), lambda i: (0, 0)),
        ],
        out_specs=[
            pl.BlockSpec((2 * _N, _LANES),
                         lambda i: (0, jnp.minimum(i, _B // _LANES - 1))),
            pl.BlockSpec((1, _B), lambda i: (0, 0)),
        ],
        out_shape=[
            jax.ShapeDtypeStruct((2 * _N, _B), jnp.float32),
            jax.ShapeDtypeStruct((1, _B), jnp.float32),
        ],
        scratch_shapes=[pltpu.VMEM((_ROWS, _B), jnp.float32),
                        pltpu.VMEM((_E, 1), jnp.float32)],
    )(xt, wt, b_embed.reshape(1, _E), W_val.T,
      param.reshape(1, 2), b_val.reshape(1, 1))
    return actt.T, val.reshape(_B)
